# Initial kernel scaffold; baseline (speedup 1.0000x reference)
import jax
import jax.numpy as jnp
from jax.experimental import pallas as pl


def kernel(x, adj_rows, adj_cols, adj_vals, W, b):
    raise NotImplementedError("write your pallas kernel here")


# speedup vs baseline: 1.7110x; 1.7110x over previous
import functools

import jax
import jax.numpy as jnp
from jax import lax
from jax.experimental import pallas as pl
from jax.experimental.pallas import tpu as pltpu
from jax.experimental.pallas import tpu_sc as plsc

B, T, N, FIN, FOUT, E = 2, 12, 10000, 128, 128, 320000
NBT = B * T
NC, NS = 2, 16
SLICES_PER_CORE = NBT // NC
NP = 10240
ROWS_PER_TILE = NP // NS
CH = 128
NCHUNK = ROWS_PER_TILE // CH
G = 128
EP_TILE = 20096
NB = EP_TILE // G
EPAD = EP_TILE * NS
NBTOT = EPAD // G
LANES = 16


def _mm_body(x_ref, wt_ref, b_ref, o_ref):
    o_ref[...] = (
        jnp.dot(x_ref[...], wt_ref[...], preferred_element_type=jnp.float32)
        + b_ref[...]
    )


def _linear(x_flat, Wt, b_row):
    rows = x_flat.shape[0]
    blk = 1000
    grid = rows // blk
    return pl.pallas_call(
        _mm_body,
        grid=(grid,),
        in_specs=[
            pl.BlockSpec((blk, FIN), lambda i: (i, 0)),
            pl.BlockSpec((FIN, FOUT), lambda i: (0, 0)),
            pl.BlockSpec((1, FOUT), lambda i: (0, 0)),
        ],
        out_specs=pl.BlockSpec((blk, FOUT), lambda i: (i, 0)),
        out_shape=jax.ShapeDtypeStruct((rows, FOUT), jnp.float32),
    )(x_flat, Wt, b_row)


def _fill_zeros(buf, rows):
    def _zrow(r, _):
        for k in range(FOUT // LANES):
            buf[r, pl.ds(k * LANES, LANES)] = jnp.zeros((LANES,), jnp.float32)
        return _
    lax.fori_loop(0, rows, _zrow, None)


def _sc_body(h_hbm, rows_hbm, cols_hbm, vals_hbm, out_hbm,
             gidx, ridx, vbuf, gbuf, rbuf, accum, sem):
    cid = lax.axis_index("c")
    sid = lax.axis_index("s")
    row0 = sid * ROWS_PER_TILE

    _fill_zeros(rbuf, CH)
    for j in range(NCHUNK):
        pltpu.sync_copy(rbuf, accum.at[pl.ds(row0 + j * CH, CH)])
    plsc.subcore_barrier()

    def _slice_step(s, _):
        c = cid * SLICES_PER_CORE + s
        hbase = c * N
        obase = c * NP

        def _batch(bi, _):
            eb = (sid * NB + bi) * G
            pltpu.sync_copy(rows_hbm.at[pl.ds(eb, G)], ridx)
            pltpu.sync_copy(cols_hbm.at[pl.ds(eb, G)], gidx)
            pltpu.sync_copy(vals_hbm.at[pl.ds(eb, G)], vbuf)

            def _gi(k, _):
                sl = pl.ds(k * LANES, LANES)
                gidx[sl] = gidx[sl] + hbase
                return _
            lax.fori_loop(0, G // LANES, _gi, None)

            pltpu.async_copy(h_hbm.at[gidx], gbuf, sem).wait()

            def _scale16(gg, _):
                g0 = gg * LANES
                v16 = vbuf[pl.ds(g0, LANES)]
                for g2 in range(LANES):
                    v = v16[g2]
                    for k in range(FOUT // LANES):
                        sl = pl.ds(k * LANES, LANES)
                        gbuf[g0 + g2, sl] = gbuf[g0 + g2, sl] * v
                return _
            lax.fori_loop(0, G // LANES, _scale16, None)

            pltpu.sync_copy(gbuf, accum.at[ridx], add=True)
            return _
        lax.fori_loop(0, NB, _batch, None)
        plsc.subcore_barrier()

        for j in range(NCHUNK):
            r0 = row0 + j * CH
            pltpu.sync_copy(accum.at[pl.ds(r0, CH)], rbuf)

            def _relu(r, _):
                for k in range(FOUT // LANES):
                    sl = pl.ds(k * LANES, LANES)
                    rbuf[r, sl] = jnp.maximum(rbuf[r, sl], 0.0)
                return _
            lax.fori_loop(0, CH, _relu, None)
            pltpu.sync_copy(rbuf, out_hbm.at[pl.ds(obase + r0, CH)])
            _fill_zeros(rbuf, CH)
            pltpu.sync_copy(rbuf, accum.at[pl.ds(r0, CH)])
        plsc.subcore_barrier()
        return _

    lax.fori_loop(0, SLICES_PER_CORE, _slice_step, None)


_sc_aggregate = functools.partial(
    pl.kernel,
    out_type=jax.ShapeDtypeStruct((NBT * NP, FOUT), jnp.float32),
    mesh=plsc.VectorSubcoreMesh(core_axis_name="c", subcore_axis_name="s"),
    scratch_types=[
        pltpu.VMEM((G,), jnp.int32),
        pltpu.VMEM((G,), jnp.int32),
        pltpu.VMEM((G,), jnp.float32),
        pltpu.VMEM((G, FOUT), jnp.float32),
        pltpu.VMEM((CH, FOUT), jnp.float32),
        pltpu.VMEM_SHARED((NP, FOUT), jnp.float32),
        pltpu.SemaphoreType.DMA,
    ],
)(_sc_body)


def kernel(x, adj_rows, adj_cols, adj_vals, W, b):
    x_flat = x.reshape(B * T * N, FIN)
    h = _linear(x_flat, W.T, b.reshape(1, FOUT))

    pad = EPAD - E
    rows_p = jnp.pad(adj_rows, (0, pad))
    cols_p = jnp.pad(adj_cols, (0, pad))
    vals_p = jnp.pad(adj_vals, (0, pad))

    out_flat = _sc_aggregate(h, rows_p, cols_p, vals_p)
    return out_flat.reshape(NBT, NP, FOUT)[:, :N].reshape(B, T, N, FOUT)


# speedup vs baseline: 3.0145x; 1.7618x over previous
import functools

import jax
import jax.numpy as jnp
from jax import lax
from jax.experimental import pallas as pl
from jax.experimental.pallas import tpu as pltpu
from jax.experimental.pallas import tpu_sc as plsc

B, T, N, FIN, FOUT, E = 2, 12, 10000, 128, 128, 320000
NBT = B * T
NC, NS = 2, 16
SLICES_PER_CORE = NBT // NC
NP = 10240
ROWS_PER_TILE = NP // NS
G = 112
NB = 180
EP_TILE = NB * G
EPAD = EP_TILE * NS
LANES = 16
NE = 4
NGB = 3
STEPS = 12
ROUNDS = NB // STEPS
CHUNKS = [(0, G), (G, G), (2 * G, G), (3 * G, G), (4 * G, G), (5 * G, 80)]


def _mm_body(x_ref, wt_ref, b_ref, o_ref):
    o_ref[...] = (
        jnp.dot(x_ref[...], wt_ref[...], preferred_element_type=jnp.float32)
        + b_ref[...]
    )


def _linear(x_flat, Wt, b_row):
    rows = x_flat.shape[0]
    blk = 1000
    grid = rows // blk
    return pl.pallas_call(
        _mm_body,
        grid=(grid,),
        in_specs=[
            pl.BlockSpec((blk, FIN), lambda i: (i, 0)),
            pl.BlockSpec((FIN, FOUT), lambda i: (0, 0)),
            pl.BlockSpec((1, FOUT), lambda i: (0, 0)),
        ],
        out_specs=pl.BlockSpec((blk, FOUT), lambda i: (i, 0)),
        out_shape=jax.ShapeDtypeStruct((rows, FOUT), jnp.float32),
    )(x_flat, Wt, b_row)


def _sc_body(h_hbm, rows_hbm, cols_hbm, vals_hbm, out_hbm, *scr):
    ridx = scr[0:NE]
    gidx = scr[NE:2 * NE]
    vbuf = scr[2 * NE:3 * NE]
    gbuf = scr[3 * NE:3 * NE + NGB]
    esem = scr[3 * NE + NGB:4 * NE + NGB]
    gsem = scr[4 * NE + NGB:4 * NE + 2 * NGB]
    ssem = scr[4 * NE + 2 * NGB:4 * NE + 3 * NGB]
    accum = scr[4 * NE + 3 * NGB]

    cid = lax.axis_index("c")
    sid = lax.axis_index("s")
    row0 = sid * ROWS_PER_TILE
    ebase = sid * EP_TILE

    def _issue_eload(e, b):
        eb = ebase + b * G
        pltpu.async_copy(rows_hbm.at[pl.ds(eb, G)], ridx[e], esem[e])
        pltpu.async_copy(cols_hbm.at[pl.ds(eb, G)], gidx[e], esem[e])
        pltpu.async_copy(vals_hbm.at[pl.ds(eb, G)], vbuf[e], esem[e])

    def _wait_eload(e):
        pltpu.make_async_copy(rows_hbm.at[pl.ds(0, G)], ridx[e], esem[e]).wait()
        pltpu.make_async_copy(cols_hbm.at[pl.ds(0, G)], gidx[e], esem[e]).wait()
        pltpu.make_async_copy(vals_hbm.at[pl.ds(0, G)], vbuf[e], esem[e]).wait()

    def _wait_gather(a):
        pltpu.make_async_copy(h_hbm.at[pl.ds(0, G)], gbuf[a], gsem[a]).wait()

    def _wait_scatter(a):
        pltpu.make_async_copy(gbuf[a], accum.at[pl.ds(0, G)], ssem[a]).wait()

    def _add_hbase(e, hbase):
        def _gi(kk, _):
            sl = pl.ds(kk * LANES, LANES)
            gidx[e][sl] = gidx[e][sl] + hbase
            return _
        lax.fori_loop(0, G // LANES, _gi, None)

    def _scale(a, e):
        def _scale16(gg, _):
            g0 = gg * LANES
            v16 = vbuf[e][pl.ds(g0, LANES)]
            for g2 in range(LANES):
                v = v16[g2]
                for kk in range(FOUT // LANES):
                    sl = pl.ds(kk * LANES, LANES)
                    gbuf[a][g0 + g2, sl] = gbuf[a][g0 + g2, sl] * v
            return _
        lax.fori_loop(0, G // LANES, _scale16, None)

    def _zero_gbuf0(rows):
        def _zrow(r, _):
            for kk in range(FOUT // LANES):
                gbuf[0][r, pl.ds(kk * LANES, LANES)] = (
                    jnp.zeros((LANES,), jnp.float32))
            return _
        lax.fori_loop(0, rows, _zrow, None)

    def _zero_accum_rows():
        for (r0, ch) in CHUNKS:
            pltpu.async_copy(gbuf[0].at[pl.ds(0, ch)],
                             accum.at[pl.ds(row0 + r0, ch)], ssem[0])
        for (r0, ch) in CHUNKS:
            pltpu.make_async_copy(gbuf[0].at[pl.ds(0, ch)],
                                  accum.at[pl.ds(row0 + r0, ch)],
                                  ssem[0]).wait()

    _zero_gbuf0(G)
    _zero_accum_rows()
    plsc.subcore_barrier()

    def _slice_step(s, _):
        c = cid * SLICES_PER_CORE + s
        hbase = c * N
        obase = c * NP

        for b0 in range(3):
            _issue_eload(b0, b0)
        _wait_eload(0)
        _add_hbase(0, hbase)
        pltpu.async_copy(h_hbm.at[gidx[0]], gbuf[0], gsem[0])

        def _half(r, jj):
            k = r * STEPS + jj
            a = jj % NGB
            e = jj % NE
            an = (jj + 1) % NGB
            en = (jj + 1) % NE
            ap = (jj - 1) % NGB
            ep = (jj + 3) % NE

            _wait_gather(a)
            @pl.when(k < NB - 1)
            def _():
                _wait_eload(en)
                _add_hbase(en, hbase)
                pltpu.async_copy(h_hbm.at[gidx[en]], gbuf[an], gsem[an])
            _scale(a, e)
            pltpu.async_copy(gbuf[a], accum.at[ridx[e]], ssem[a], add=True)
            @pl.when(k >= 1)
            def _():
                _wait_scatter(ap)
            @pl.when(k <= NB - 4)
            def _():
                _issue_eload(ep, k + 3)

        def _round(r, _):
            for jj in range(STEPS):
                _half(r, jj)
            return _
        lax.fori_loop(0, ROUNDS, _round, None)
        _wait_scatter((NB - 1) % NGB)
        plsc.subcore_barrier()

        for (r0, ch) in CHUNKS:
            pltpu.sync_copy(accum.at[pl.ds(row0 + r0, ch)],
                            gbuf[0].at[pl.ds(0, ch)])

            def _relu(r, _):
                for kk in range(FOUT // LANES):
                    sl = pl.ds(kk * LANES, LANES)
                    gbuf[0][r, sl] = jnp.maximum(gbuf[0][r, sl], 0.0)
                return _
            lax.fori_loop(0, ch, _relu, None)
            pltpu.sync_copy(gbuf[0].at[pl.ds(0, ch)],
                            out_hbm.at[pl.ds(obase + row0 + r0, ch)])

        _zero_gbuf0(G)
        _zero_accum_rows()
        plsc.subcore_barrier()
        return _

    lax.fori_loop(0, SLICES_PER_CORE, _slice_step, None)


_sc_aggregate = functools.partial(
    pl.kernel,
    out_type=jax.ShapeDtypeStruct((NBT * NP, FOUT), jnp.float32),
    mesh=plsc.VectorSubcoreMesh(core_axis_name="c", subcore_axis_name="s"),
    scratch_types=(
        [pltpu.VMEM((G,), jnp.int32) for _ in range(NE)]
        + [pltpu.VMEM((G,), jnp.int32) for _ in range(NE)]
        + [pltpu.VMEM((G,), jnp.float32) for _ in range(NE)]
        + [pltpu.VMEM((G, FOUT), jnp.float32) for _ in range(NGB)]
        + [pltpu.SemaphoreType.DMA for _ in range(NE + 2 * NGB)]
        + [pltpu.VMEM_SHARED((NP, FOUT), jnp.float32)]
    ),
)(_sc_body)


def kernel(x, adj_rows, adj_cols, adj_vals, W, b):
    x_flat = x.reshape(B * T * N, FIN)
    h = _linear(x_flat, W.T, b.reshape(1, FOUT))

    pad = EPAD - E
    rows_p = jnp.pad(adj_rows, (0, pad))
    cols_p = jnp.pad(adj_cols, (0, pad))
    vals_p = jnp.pad(adj_vals, (0, pad))

    out_flat = _sc_aggregate(h, rows_p, cols_p, vals_p)
    return out_flat.reshape(NBT, NP, FOUT)[:, :N].reshape(B, T, N, FOUT)


# speedup vs baseline: 3.0397x; 1.0083x over previous
import functools

import jax
import jax.numpy as jnp
from jax import lax
from jax.experimental import pallas as pl
from jax.experimental.pallas import tpu as pltpu
from jax.experimental.pallas import tpu_sc as plsc

B, T, N, FIN, FOUT, E = 2, 12, 10000, 128, 128, 320000
NBT = B * T
NC, NS = 2, 16
SLICES_PER_CORE = NBT // NC
NP = 10240
ROWS_PER_TILE = NP // NS
G = 112
NB = 180
EP_TILE = NB * G
EPAD = EP_TILE * NS
LANES = 16
NE = 4
NGB = 3
STEPS = 12
ROUNDS = NB // STEPS
CHUNKS = [(0, G), (G, G), (2 * G, G), (3 * G, G), (4 * G, G), (5 * G, 80)]


def _mm_body(x_ref, wt_ref, b_ref, o_ref):
    o_ref[...] = (
        jnp.dot(x_ref[...], wt_ref[...], preferred_element_type=jnp.float32)
        + b_ref[...]
    )


def _linear(x_flat, Wt, b_row):
    rows = x_flat.shape[0]
    blk = 1000
    grid = rows // blk
    return pl.pallas_call(
        _mm_body,
        grid=(grid,),
        in_specs=[
            pl.BlockSpec((blk, FIN), lambda i: (i, 0)),
            pl.BlockSpec((FIN, FOUT), lambda i: (0, 0)),
            pl.BlockSpec((1, FOUT), lambda i: (0, 0)),
        ],
        out_specs=pl.BlockSpec((blk, FOUT), lambda i: (i, 0)),
        out_shape=jax.ShapeDtypeStruct((rows, FOUT), jnp.float32),
    )(x_flat, Wt, b_row)


def _sc_body(h_hbm, rows_hbm, cols_hbm, vals_hbm, out_hbm, *scr):
    ridx = scr[0:NE]
    gidx = scr[NE:2 * NE]
    vbuf = scr[2 * NE:3 * NE]
    gbuf = scr[3 * NE:3 * NE + NGB]
    esem = scr[3 * NE + NGB:4 * NE + NGB]
    gsem = scr[4 * NE + NGB:4 * NE + 2 * NGB]
    ssem = scr[4 * NE + 2 * NGB:4 * NE + 3 * NGB]
    accum = scr[4 * NE + 3 * NGB]

    cid = lax.axis_index("c")
    sid = lax.axis_index("s")
    row0 = sid * ROWS_PER_TILE
    ebase = sid * EP_TILE

    def _issue_eload(e, b):
        eb = ebase + b * G
        pltpu.async_copy(rows_hbm.at[pl.ds(eb, G)], ridx[e], esem[e])
        pltpu.async_copy(cols_hbm.at[pl.ds(eb, G)], gidx[e], esem[e])
        pltpu.async_copy(vals_hbm.at[pl.ds(eb, G)], vbuf[e], esem[e])

    def _wait_eload(e):
        pltpu.make_async_copy(rows_hbm.at[pl.ds(0, G)], ridx[e], esem[e]).wait()
        pltpu.make_async_copy(cols_hbm.at[pl.ds(0, G)], gidx[e], esem[e]).wait()
        pltpu.make_async_copy(vals_hbm.at[pl.ds(0, G)], vbuf[e], esem[e]).wait()

    def _issue_gather(e, a):
        h2 = G // 2
        pltpu.async_copy(h_hbm.at[gidx[e].at[pl.ds(0, h2)]],
                         gbuf[a].at[pl.ds(0, h2)], gsem[a])
        pltpu.async_copy(h_hbm.at[gidx[e].at[pl.ds(h2, h2)]],
                         gbuf[a].at[pl.ds(h2, h2)], gsem[a])

    def _wait_gather(a):
        pltpu.make_async_copy(h_hbm.at[pl.ds(0, G)], gbuf[a], gsem[a]).wait()

    def _wait_scatter(a):
        pltpu.make_async_copy(gbuf[a], accum.at[pl.ds(0, G)], ssem[a]).wait()

    def _add_hbase(e, hbase):
        def _gi(kk, _):
            sl = pl.ds(kk * LANES, LANES)
            gidx[e][sl] = gidx[e][sl] + hbase
            return _
        lax.fori_loop(0, G // LANES, _gi, None)

    def _scale(a, e):
        def _scale16(gg, _):
            g0 = gg * LANES
            v16 = vbuf[e][pl.ds(g0, LANES)]
            for g2 in range(LANES):
                v = v16[g2]
                for kk in range(FOUT // LANES):
                    sl = pl.ds(kk * LANES, LANES)
                    gbuf[a][g0 + g2, sl] = gbuf[a][g0 + g2, sl] * v
            return _
        lax.fori_loop(0, G // LANES, _scale16, None)

    def _zero_gbuf0(rows):
        def _zrow(r, _):
            for kk in range(FOUT // LANES):
                gbuf[0][r, pl.ds(kk * LANES, LANES)] = (
                    jnp.zeros((LANES,), jnp.float32))
            return _
        lax.fori_loop(0, rows, _zrow, None)

    def _zero_accum_rows():
        for (r0, ch) in CHUNKS:
            pltpu.async_copy(gbuf[0].at[pl.ds(0, ch)],
                             accum.at[pl.ds(row0 + r0, ch)], ssem[0])
        for (r0, ch) in CHUNKS:
            pltpu.make_async_copy(gbuf[0].at[pl.ds(0, ch)],
                                  accum.at[pl.ds(row0 + r0, ch)],
                                  ssem[0]).wait()

    _zero_gbuf0(G)
    _zero_accum_rows()
    plsc.subcore_barrier()

    def _slice_step(s, _):
        c = cid * SLICES_PER_CORE + s
        hbase = c * N
        obase = c * NP

        for b0 in range(3):
            _issue_eload(b0, b0)
        _wait_eload(0)
        _add_hbase(0, hbase)
        _issue_gather(0, 0)

        def _half(r, jj):
            k = r * STEPS + jj
            a = jj % NGB
            e = jj % NE
            an = (jj + 1) % NGB
            en = (jj + 1) % NE
            ap = (jj - 1) % NGB
            ep = (jj + 3) % NE

            _wait_gather(a)
            @pl.when(k < NB - 1)
            def _():
                _wait_eload(en)
                _add_hbase(en, hbase)
                _issue_gather(en, an)
            _scale(a, e)
            pltpu.async_copy(gbuf[a], accum.at[ridx[e]], ssem[a], add=True)
            @pl.when(k >= 1)
            def _():
                _wait_scatter(ap)
            @pl.when(k <= NB - 4)
            def _():
                _issue_eload(ep, k + 3)

        def _round(r, _):
            for jj in range(STEPS):
                _half(r, jj)
            return _
        lax.fori_loop(0, ROUNDS, _round, None)
        _wait_scatter((NB - 1) % NGB)
        plsc.subcore_barrier()

        for (r0, ch) in CHUNKS:
            pltpu.sync_copy(accum.at[pl.ds(row0 + r0, ch)],
                            gbuf[0].at[pl.ds(0, ch)])

            def _relu(r, _):
                for kk in range(FOUT // LANES):
                    sl = pl.ds(kk * LANES, LANES)
                    gbuf[0][r, sl] = jnp.maximum(gbuf[0][r, sl], 0.0)
                return _
            lax.fori_loop(0, ch, _relu, None)
            pltpu.sync_copy(gbuf[0].at[pl.ds(0, ch)],
                            out_hbm.at[pl.ds(obase + row0 + r0, ch)])

        _zero_gbuf0(G)
        _zero_accum_rows()
        plsc.subcore_barrier()
        return _

    lax.fori_loop(0, SLICES_PER_CORE, _slice_step, None)


_sc_aggregate = functools.partial(
    pl.kernel,
    out_type=jax.ShapeDtypeStruct((NBT * NP, FOUT), jnp.float32),
    mesh=plsc.VectorSubcoreMesh(core_axis_name="c", subcore_axis_name="s"),
    scratch_types=(
        [pltpu.VMEM((G,), jnp.int32) for _ in range(NE)]
        + [pltpu.VMEM((G,), jnp.int32) for _ in range(NE)]
        + [pltpu.VMEM((G,), jnp.float32) for _ in range(NE)]
        + [pltpu.VMEM((G, FOUT), jnp.float32) for _ in range(NGB)]
        + [pltpu.SemaphoreType.DMA for _ in range(NE + 2 * NGB)]
        + [pltpu.VMEM_SHARED((NP, FOUT), jnp.float32)]
    ),
)(_sc_body)


def kernel(x, adj_rows, adj_cols, adj_vals, W, b):
    x_flat = x.reshape(B * T * N, FIN)
    h = _linear(x_flat, W.T, b.reshape(1, FOUT))

    pad = EPAD - E
    rows_p = jnp.pad(adj_rows, (0, pad))
    cols_p = jnp.pad(adj_cols, (0, pad))
    vals_p = jnp.pad(adj_vals, (0, pad))

    out_flat = _sc_aggregate(h, rows_p, cols_p, vals_p)
    return out_flat.reshape(NBT, NP, FOUT)[:, :N].reshape(B, T, N, FOUT)
